# baseline (speedup 1.0000x reference)
import jax
import jax.numpy as jnp
from jax.experimental import pallas as pl

_ROWS = 1024 * 128
_C = 192
_BLOCK_ROWS = 1024


def _permute_body(x_ref, o_ref):
    x = x_ref[...]
    a = x[:, 48:96]
    b = x[:, 96:144]
    mid = jnp.stack([a, b], axis=-1).reshape(x.shape[0], 96)
    o_ref[...] = jnp.concatenate([x[:, :48], mid, x[:, 144:]], axis=1)


def kernel(inputs):
    x = inputs.reshape(_ROWS, _C)
    out = pl.pallas_call(
        _permute_body,
        grid=(_ROWS // _BLOCK_ROWS,),
        in_specs=[pl.BlockSpec((_BLOCK_ROWS, _C), lambda i: (i, 0))],
        out_specs=pl.BlockSpec((_BLOCK_ROWS, _C), lambda i: (i, 0)),
        out_shape=jax.ShapeDtypeStruct((_ROWS, _C), jnp.float32),
    )(x)
    return out.reshape(inputs.shape)


# speedup vs baseline: 8.8619x; 8.8619x over previous
import numpy as np
import jax
import jax.numpy as jnp
from jax.experimental import pallas as pl

_ROWS = 1024 * 128
_C = 192
_BLOCK_ROWS = 1024


def _perm() -> np.ndarray:
    mixed = np.stack([np.arange(48, 96), np.arange(96, 144)]).T.reshape(-1)
    return np.concatenate([np.arange(0, 48), mixed, np.arange(144, 192)])


def _perm_matrix() -> np.ndarray:
    p = np.zeros((_C, _C), dtype=np.float32)
    p[_perm(), np.arange(_C)] = 1.0
    return p


def _permute_body(x_ref, p_ref, o_ref):
    o_ref[...] = jnp.dot(x_ref[...], p_ref[...],
                         preferred_element_type=jnp.float32)


def kernel(inputs):
    x = inputs.reshape(_ROWS, _C)
    p = jnp.asarray(_perm_matrix())
    out = pl.pallas_call(
        _permute_body,
        grid=(_ROWS // _BLOCK_ROWS,),
        in_specs=[
            pl.BlockSpec((_BLOCK_ROWS, _C), lambda i: (i, 0)),
            pl.BlockSpec((_C, _C), lambda i: (0, 0)),
        ],
        out_specs=pl.BlockSpec((_BLOCK_ROWS, _C), lambda i: (i, 0)),
        out_shape=jax.ShapeDtypeStruct((_ROWS, _C), jnp.float32),
    )(x, p)
    return out.reshape(inputs.shape)
